# Initial kernel scaffold; baseline (speedup 1.0000x reference)
import jax
import jax.numpy as jnp
from jax.experimental import pallas as pl


def kernel(inputs, W_lin, b_lin, gamma, beta, W_s, b_s):
    raise NotImplementedError("write your pallas kernel here")


# speedup vs baseline: 24.2762x; 24.2762x over previous
import jax
import jax.numpy as jnp
from jax.experimental import pallas as pl
from jax.experimental.pallas import tpu as pltpu

IN_CH = 3
OUT_CH = 128
BATCH = 256
TOTAL = 2048
TOPK = 50
EPS = 1e-5
NCHUNK = TOTAL // BATCH
DU = 8
ND = OUT_CH // DU


def _bn_kernel(x_ref, w_ref, b_ref, g_ref, be_ref, o_ref):
    x = x_ref[0].astype(jnp.bfloat16).astype(jnp.float32)
    w = w_ref[...].astype(jnp.bfloat16).astype(jnp.float32)
    out = jax.lax.dot_general(x, w, (((1,), (1,)), ((), ())),
                              preferred_element_type=jnp.float32,
                              precision=jax.lax.Precision.HIGHEST)
    out = out + b_ref[...]
    mu = jnp.mean(out, axis=0, keepdims=True)
    var = jnp.mean((out - mu) ** 2, axis=0, keepdims=True)
    o_ref[0] = (out - mu) / jnp.sqrt(var + EPS) * g_ref[...] + be_ref[...]


def _score_kernel(xti_ref, xt_ref, w_ref, bs_ref, s_ref):
    d0 = pl.program_id(1)

    @pl.when(d0 == 0)
    def _init():
        s_ref[...] = jnp.zeros((BATCH, TOTAL), jnp.float32)

    ident = (jax.lax.broadcasted_iota(jnp.int32, (DU, DU), 0) ==
             jax.lax.broadcasted_iota(jnp.int32, (DU, DU), 1)).astype(jnp.float32)
    xi_cols = jax.lax.dot_general(xti_ref[...], ident, (((0,), (0,)), ((), ())),
                                  preferred_element_type=jnp.float32,
                                  precision=jax.lax.Precision.HIGHEST)

    acc = s_ref[...]
    for k in range(DU):
        xi = xi_cols[:, k:k + 1]
        xn = xt_ref[k:k + 1, :]
        d2 = (xi * xi + xn * xn) - 2.0 * (xi * xn)
        r = d2.astype(jnp.bfloat16).astype(jnp.float32)
        acc = acc + r * w_ref[0, d0 * DU + k]
    s_ref[...] = acc

    @pl.when(d0 == ND - 1)
    def _finish():
        scores = jnp.maximum(s_ref[...] + bs_ref[0, 0], 0.0)
        bits = jax.lax.bitcast_convert_type(scores, jnp.int32)

        def body(_, carry):
            lo, hi = carry
            mid = lo + ((hi - lo) >> 1)
            cnt = jnp.sum(jnp.where(bits >= mid, 1, 0), axis=1, keepdims=True)
            ok = cnt >= TOPK
            return jnp.where(ok, mid, lo), jnp.where(ok, hi, mid)

        lo0 = jnp.zeros((BATCH, 1), jnp.int32)
        hi0 = jnp.full((BATCH, 1), 0x7F800000, jnp.int32)
        lo, _ = jax.lax.fori_loop(0, 31, body, (lo0, hi0))

        n_gt = jnp.sum(jnp.where(bits > lo, 1, 0), axis=1, keepdims=True)
        need = TOPK - n_gt
        eq = bits == lo
        c = jnp.where(eq, 1, 0)
        p = c
        sh = 1
        while sh < TOTAL:
            p = p + jnp.concatenate(
                [jnp.zeros((BATCH, sh), jnp.int32), p[:, :TOTAL - sh]], axis=1)
            sh *= 2
        sel = eq & ((p - c) < need)

        mask = (bits > lo) | sel
        m = jnp.max(scores, axis=1, keepdims=True)
        e = jnp.where(mask, jnp.exp(scores - m), 0.0)
        z = jnp.sum(e, axis=1, keepdims=True)
        s_ref[...] = e / z


def kernel(inputs, W_lin, b_lin, gamma, beta, W_s, b_s):
    flat = inputs.reshape(NCHUNK, BATCH, -1)
    outputs = pl.pallas_call(
        _bn_kernel,
        grid=(NCHUNK,),
        in_specs=[
            pl.BlockSpec((1, BATCH, flat.shape[-1]), lambda i: (i, 0, 0)),
            pl.BlockSpec((OUT_CH, flat.shape[-1]), lambda i: (0, 0)),
            pl.BlockSpec((1, OUT_CH), lambda i: (0, 0)),
            pl.BlockSpec((1, OUT_CH), lambda i: (0, 0)),
            pl.BlockSpec((1, OUT_CH), lambda i: (0, 0)),
        ],
        out_specs=pl.BlockSpec((1, BATCH, OUT_CH), lambda i: (i, 0, 0)),
        out_shape=jax.ShapeDtypeStruct((NCHUNK, BATCH, OUT_CH), jnp.float32),
    )(flat, W_lin, b_lin[None, :], gamma[None, :], beta[None, :])
    outputs = outputs.reshape(TOTAL, OUT_CH)

    outputs_t = outputs.T
    w_rounded = W_s.astype(jnp.bfloat16).astype(jnp.float32)

    S = pl.pallas_call(
        _score_kernel,
        grid=(NCHUNK, ND),
        in_specs=[
            pl.BlockSpec((DU, BATCH), lambda i, d: (d, i)),
            pl.BlockSpec((DU, TOTAL), lambda i, d: (d, 0)),
            pl.BlockSpec(memory_space=pltpu.SMEM),
            pl.BlockSpec(memory_space=pltpu.SMEM),
        ],
        out_specs=pl.BlockSpec((BATCH, TOTAL), lambda i, d: (i, 0)),
        out_shape=jax.ShapeDtypeStruct((TOTAL, TOTAL), jnp.float32),
    )(outputs_t, outputs_t, w_rounded, b_s.reshape(1, 1))
    return outputs, S
